# baseline (speedup 1.0000x reference)
import functools

import jax
import jax.numpy as jnp
from jax import lax
from jax.experimental import pallas as pl
from jax.experimental.pallas import tpu as pltpu
from jax.experimental.pallas import tpu_sc as plsc

N = 10000
E = 320000
D = 128
G = 64

NC = 2
NS = 16
NW = NC * NS
EC = 96
KBUF = 3
CHUNKS = 105
N_ACC = 10112
ROWS_ACC = N_ACC // NS
E_PAD = NW * EC * CHUNKS
N_PAD = 10112
ROWS_PER = N_PAD // NS
BN = 128
GRID = N_PAD // BN

def _deg_body(dstp_hbm, zeros1_hbm, out_hbm, dst_v, ones_v, acc):
    c = lax.axis_index("c")
    s = lax.axis_index("s")
    wid = c * NS + s
    for j in range(EC // 16):
        ones_v[pl.ds(j * 16, 16)] = jnp.ones((16,), jnp.float32)

    @pl.when(s == 0)
    def _():
        pltpu.sync_copy(zeros1_hbm, acc)

    pltpu.sync_copy(dstp_hbm.at[wid], dst_v)
    plsc.subcore_barrier()

    def body(i, carry):
        pltpu.sync_copy(ones_v, acc.at[dst_v.at[i]], add=True)
        return carry

    lax.fori_loop(0, CHUNKS, body, 0)
    plsc.subcore_barrier()

    @pl.when(s == 0)
    def _():
        pltpu.sync_copy(acc, out_hbm.at[pl.ds(c * N_PAD, N_PAD)])


def _agg_body(hp_hbm, pk_hbm, zeros2_hbm, out_hbm,
              pk_v, src0, dst0, src1, dst1, src2, dst2,
              rows0, rows1, rows2, acc, sem0, sem1, sem2):
    c = lax.axis_index("c")
    s = lax.axis_index("s")
    wid = c * NS + s
    rows = (rows0, rows1, rows2)
    sems = (sem0, sem1, sem2)
    srcs = (src0, src1, src2)
    dsts = (dst0, dst1, dst2)
    pltpu.sync_copy(zeros2_hbm, acc.at[pl.ds(s * ROWS_ACC, ROWS_ACC)])
    pltpu.sync_copy(pk_hbm.at[wid], pk_v)
    plsc.subcore_barrier()

    def dec(i, j):
        for k in range(EC // 16):
            v = pk_v[pl.ds(i * EC + k * 16, 16)]
            srcs[j][pl.ds(k * 16, 16)] = lax.shift_right_logical(v, 15)
            dsts[j][pl.ds(k * 16, 16)] = lax.bitwise_and(v, 32767)

    def gather(j):
        return pltpu.make_async_copy(hp_hbm.at[srcs[j]], rows[j], sems[j])

    for j in range(KBUF):
        dec(j, j)
        gather(j).start()

    def body(t, carry):
        i0 = KBUF * t
        for j in range(KBUF):
            i = i0 + j
            gather(j).wait()
            pltpu.sync_copy(rows[j], acc.at[dsts[j]], add=True)

            @pl.when(t < CHUNKS // KBUF - 1)
            def _():
                dec(i + KBUF, j)
                gather(j).start()

        return carry

    lax.fori_loop(0, CHUNKS // KBUF, body, 0)
    plsc.subcore_barrier()
    pltpu.sync_copy(acc.at[pl.ds(s * ROWS_ACC, ROWS_ACC)],
                    out_hbm.at[c, pl.ds(s * ROWS_ACC, ROWS_ACC)])


@functools.lru_cache(maxsize=None)
def _sc_kernels():
    mesh = plsc.VectorSubcoreMesh(core_axis_name="c", subcore_axis_name="s")
    deg = pl.kernel(
        _deg_body,
        out_type=jax.ShapeDtypeStruct((NC * N_PAD,), jnp.float32),
        mesh=mesh,
        scratch_types=[
            pltpu.VMEM((CHUNKS, EC), jnp.int32),
            pltpu.VMEM((EC,), jnp.float32),
            pltpu.VMEM_SHARED((N_PAD,), jnp.float32),
        ],
    )
    agg = pl.kernel(
        _agg_body,
        out_type=jax.ShapeDtypeStruct((NC, N_ACC, D), jnp.float32),
        mesh=mesh,
        scratch_types=[
            pltpu.VMEM((CHUNKS * EC,), jnp.int32),
            pltpu.VMEM((EC,), jnp.int32),
            pltpu.VMEM((EC,), jnp.int32),
            pltpu.VMEM((EC,), jnp.int32),
            pltpu.VMEM((EC,), jnp.int32),
            pltpu.VMEM((EC,), jnp.int32),
            pltpu.VMEM((EC,), jnp.int32),
            pltpu.VMEM((EC, D), jnp.float32),
            pltpu.VMEM((EC, D), jnp.float32),
            pltpu.VMEM((EC, D), jnp.float32),
            pltpu.VMEM_SHARED((N_ACC, D), jnp.float32),
            pltpu.SemaphoreType.DMA,
            pltpu.SemaphoreType.DMA,
            pltpu.SemaphoreType.DMA,
        ],
    )
    return deg, agg


def _proj_body(x_ref, w_ref, d0_ref, d1_ref, o_ref):
    dinv = lax.rsqrt(d0_ref[...] + d1_ref[...] + 1.0)
    h = jnp.dot(x_ref[...], w_ref[...], preferred_element_type=jnp.float32)
    o_ref[...] = dinv * h


def _mid_body(a0_ref, a1_ref, hp_ref, d0_ref, d1_ref, b_ref, w_ref, o_ref):
    dinv = lax.rsqrt(d0_ref[...] + d1_ref[...] + 1.0)
    act = jnp.maximum(
        dinv * (a0_ref[...] + a1_ref[...] + hp_ref[...]) + b_ref[...], 0.0)
    o_ref[...] = dinv * jnp.dot(act, w_ref[...],
                                preferred_element_type=jnp.float32)


def _final_body(a0_ref, a1_ref, hp_ref, d0_ref, d1_ref, b_ref, bt_ref,
                o_ref, sums, cnt):
    i = pl.program_id(0)

    @pl.when(i == 0)
    def _():
        sums[...] = jnp.zeros_like(sums)
        cnt[...] = jnp.zeros_like(cnt)

    dinv = lax.rsqrt(d0_ref[...] + d1_ref[...] + 1.0)
    h3 = dinv * (a0_ref[...] + a1_ref[...] + hp_ref[...]) + b_ref[...]
    oh = (bt_ref[...] == lax.broadcasted_iota(jnp.int32, (BN, G), 1)
          ).astype(jnp.float32)
    dn = (((0,), (0,)), ((), ()))
    sums[...] += lax.dot_general(oh, h3, dn,
                                 preferred_element_type=jnp.float32)
    cnt[...] += lax.dot_general(oh, jnp.ones((BN, D), jnp.float32), dn,
                                preferred_element_type=jnp.float32)

    @pl.when(i == pl.num_programs(0) - 1)
    def _():
        o_ref[...] = sums[...] / jnp.maximum(cnt[...], 1.0)


_row = pl.BlockSpec((BN, D), lambda i: (i, 0))
_col = pl.BlockSpec((BN, 1), lambda i: (i, 0))
_mat = pl.BlockSpec((D, D), lambda i: (0, 0))
_vec = pl.BlockSpec((1, D), lambda i: (0, 0))

_proj_tc = pl.pallas_call(
    _proj_body,
    grid=(GRID,),
    in_specs=[_row, _mat, _col, _col],
    out_specs=_row,
    out_shape=jax.ShapeDtypeStruct((N_PAD, D), jnp.float32),
)

_mid_tc = pl.pallas_call(
    _mid_body,
    grid=(GRID,),
    in_specs=[_row, _row, _row, _col, _col, _vec, _mat],
    out_specs=_row,
    out_shape=jax.ShapeDtypeStruct((N_PAD, D), jnp.float32),
)

_final_tc = pl.pallas_call(
    _final_body,
    grid=(GRID,),
    in_specs=[_row, _row, _row, _col, _col, _vec,
              pl.BlockSpec((BN, 1), lambda i: (i, 0))],
    out_specs=pl.BlockSpec((G, D), lambda i: (0, 0)),
    out_shape=jax.ShapeDtypeStruct((G, D), jnp.float32),
    scratch_shapes=[pltpu.VMEM((G, D), jnp.float32),
                    pltpu.VMEM((G, D), jnp.float32)],
    compiler_params=pltpu.CompilerParams(
        dimension_semantics=("arbitrary",)),
)


def kernel(x, edge_index, batch, W1, b1, W2, b2, W3, b3):
    src = edge_index[0]
    dst = edge_index[1]
    padn = E_PAD - E
    padr = N + (jnp.arange(padn, dtype=jnp.int32) % (N_ACC - N))
    packed = jnp.concatenate(
        [jnp.left_shift(src, 15) | dst,
         jnp.left_shift(padr, 15) | padr]).reshape(NW, CHUNKS * EC)
    dstp = jnp.concatenate([dst, padr]).reshape(NW, CHUNKS, EC)
    xp = jnp.pad(x, ((0, N_PAD - N), (0, 0)))
    btp = jnp.pad(batch, (0, N_PAD - N),
                  constant_values=G).reshape(N_PAD, 1)
    z1 = jnp.zeros((N_PAD,), jnp.float32)
    z2 = jnp.zeros((ROWS_ACC, D), jnp.float32)

    deg_fn, agg_fn = _sc_kernels()
    degs = deg_fn(dstp, z1)
    d0 = degs[:N_PAD].reshape(N_PAD, 1)
    d1 = degs[N_PAD:].reshape(N_PAD, 1)

    h1p = _proj_tc(xp, W1, d0, d1)
    a1 = agg_fn(h1p, packed, z2)
    h2p = _mid_tc(a1[0], a1[1], h1p, d0, d1, b1.reshape(1, D), W2)
    a2 = agg_fn(h2p, packed, z2)
    h3p = _mid_tc(a2[0], a2[1], h2p, d0, d1, b2.reshape(1, D), W3)
    a3 = agg_fn(h3p, packed, z2)
    return _final_tc(a3[0], a3[1], h3p, d0, d1, b3.reshape(1, D), btp)


# speedup vs baseline: 1.1798x; 1.1798x over previous
import functools

import jax
import jax.numpy as jnp
from jax import lax
from jax.experimental import pallas as pl
from jax.experimental.pallas import tpu as pltpu
from jax.experimental.pallas import tpu_sc as plsc

N = 10000
E = 320000
D = 128
G = 64

NC = 2
NS = 16
NW = NC * NS
EC = 96
KBUF = 3
CHUNKS = 105
N_ACC = 10112
ROWS_ACC = N_ACC // NS
E_PAD = NW * EC * CHUNKS
N_PAD = 10240
ROWS_PER = N_PAD // NS
BN = 256
GRID = N_PAD // BN

def _deg_body(dstp_hbm, zeros1_hbm, out_hbm, dst_v, ones_v, acc):
    c = lax.axis_index("c")
    s = lax.axis_index("s")
    wid = c * NS + s
    for j in range(EC // 16):
        ones_v[pl.ds(j * 16, 16)] = jnp.ones((16,), jnp.float32)
    pltpu.sync_copy(zeros1_hbm, acc.at[pl.ds(s * ROWS_PER, ROWS_PER)])
    pltpu.sync_copy(dstp_hbm.at[wid], dst_v)
    plsc.subcore_barrier()

    def body(i, carry):
        pltpu.sync_copy(ones_v, acc.at[dst_v.at[i]], add=True)
        return carry

    lax.fori_loop(0, CHUNKS, body, 0)
    plsc.subcore_barrier()
    pltpu.sync_copy(acc.at[pl.ds(s * ROWS_PER, ROWS_PER)],
                    out_hbm.at[c, pl.ds(s * ROWS_PER, ROWS_PER)])


def _agg_body(hp_hbm, pk_hbm, zeros2_hbm, out_hbm,
              pk_v, src0, dst0, src1, dst1, src2, dst2,
              rows0, rows1, rows2, acc, sem0, sem1, sem2):
    c = lax.axis_index("c")
    s = lax.axis_index("s")
    wid = c * NS + s
    rows = (rows0, rows1, rows2)
    sems = (sem0, sem1, sem2)
    srcs = (src0, src1, src2)
    dsts = (dst0, dst1, dst2)
    pltpu.sync_copy(zeros2_hbm, acc.at[pl.ds(s * ROWS_ACC, ROWS_ACC)])
    pltpu.sync_copy(pk_hbm.at[wid], pk_v)
    plsc.subcore_barrier()

    def dec(i, j):
        for k in range(EC // 16):
            v = pk_v[pl.ds(i * EC + k * 16, 16)]
            srcs[j][pl.ds(k * 16, 16)] = lax.shift_right_logical(v, 15)
            dsts[j][pl.ds(k * 16, 16)] = lax.bitwise_and(v, 32767)

    def gather(j):
        return pltpu.make_async_copy(hp_hbm.at[srcs[j]], rows[j], sems[j])

    for j in range(KBUF):
        dec(j, j)
        gather(j).start()

    def body(t, carry):
        i0 = KBUF * t
        for j in range(KBUF):
            i = i0 + j
            gather(j).wait()
            pltpu.sync_copy(rows[j], acc.at[dsts[j]], add=True)

            @pl.when(t < CHUNKS // KBUF - 1)
            def _():
                dec(i + KBUF, j)
                gather(j).start()

        return carry

    lax.fori_loop(0, CHUNKS // KBUF, body, 0)
    plsc.subcore_barrier()
    pltpu.sync_copy(acc.at[pl.ds(s * ROWS_ACC, ROWS_ACC)],
                    out_hbm.at[c, pl.ds(s * ROWS_ACC, ROWS_ACC)])


@functools.lru_cache(maxsize=None)
def _sc_kernels():
    mesh = plsc.VectorSubcoreMesh(core_axis_name="c", subcore_axis_name="s")
    deg = pl.kernel(
        _deg_body,
        out_type=jax.ShapeDtypeStruct((NC, N_PAD), jnp.float32),
        mesh=mesh,
        scratch_types=[
            pltpu.VMEM((CHUNKS, EC), jnp.int32),
            pltpu.VMEM((EC,), jnp.float32),
            pltpu.VMEM_SHARED((N_PAD,), jnp.float32),
        ],
    )
    agg = pl.kernel(
        _agg_body,
        out_type=jax.ShapeDtypeStruct((NC, N_ACC, D), jnp.float32),
        mesh=mesh,
        scratch_types=[
            pltpu.VMEM((CHUNKS * EC,), jnp.int32),
            pltpu.VMEM((EC,), jnp.int32),
            pltpu.VMEM((EC,), jnp.int32),
            pltpu.VMEM((EC,), jnp.int32),
            pltpu.VMEM((EC,), jnp.int32),
            pltpu.VMEM((EC,), jnp.int32),
            pltpu.VMEM((EC,), jnp.int32),
            pltpu.VMEM((EC, D), jnp.float32),
            pltpu.VMEM((EC, D), jnp.float32),
            pltpu.VMEM((EC, D), jnp.float32),
            pltpu.VMEM_SHARED((N_ACC, D), jnp.float32),
            pltpu.SemaphoreType.DMA,
            pltpu.SemaphoreType.DMA,
            pltpu.SemaphoreType.DMA,
        ],
    )
    return deg, agg


def _proj_body(x_ref, w_ref, d0_ref, d1_ref, o_ref):
    dinv = lax.rsqrt(d0_ref[...] + d1_ref[...] + 1.0)
    h = jnp.dot(x_ref[...], w_ref[...], preferred_element_type=jnp.float32)
    o_ref[...] = dinv * h


def _mid_body(a0_ref, a1_ref, hp_ref, d0_ref, d1_ref, b_ref, w_ref, o_ref):
    dinv = lax.rsqrt(d0_ref[...] + d1_ref[...] + 1.0)
    act = jnp.maximum(
        dinv * (a0_ref[...] + a1_ref[...] + hp_ref[...]) + b_ref[...], 0.0)
    o_ref[...] = dinv * jnp.dot(act, w_ref[...],
                                preferred_element_type=jnp.float32)


def _final_body(a0_ref, a1_ref, hp_ref, d0_ref, d1_ref, b_ref, bt_ref,
                o_ref, sums, cnt):
    i = pl.program_id(0)

    @pl.when(i == 0)
    def _():
        sums[...] = jnp.zeros_like(sums)
        cnt[...] = jnp.zeros_like(cnt)

    dinv = lax.rsqrt(d0_ref[...] + d1_ref[...] + 1.0)
    h3 = dinv * (a0_ref[...] + a1_ref[...] + hp_ref[...]) + b_ref[...]
    oh = (bt_ref[...] == lax.broadcasted_iota(jnp.int32, (BN, G), 1)
          ).astype(jnp.float32)
    dn = (((0,), (0,)), ((), ()))
    sums[...] += lax.dot_general(oh, h3, dn,
                                 preferred_element_type=jnp.float32)
    cnt[...] += lax.dot_general(oh, jnp.ones((BN, D), jnp.float32), dn,
                                preferred_element_type=jnp.float32)

    @pl.when(i == pl.num_programs(0) - 1)
    def _():
        o_ref[...] = sums[...] / jnp.maximum(cnt[...], 1.0)


_row = pl.BlockSpec((BN, D), lambda i: (i, 0))
_col = pl.BlockSpec((BN, 1), lambda i: (i, 0))
_mat = pl.BlockSpec((D, D), lambda i: (0, 0))
_vec = pl.BlockSpec((1, D), lambda i: (0, 0))

_proj_tc = pl.pallas_call(
    _proj_body,
    grid=(GRID,),
    in_specs=[_row, _mat, _col, _col],
    out_specs=_row,
    out_shape=jax.ShapeDtypeStruct((N_PAD, D), jnp.float32),
)

_mid_tc = pl.pallas_call(
    _mid_body,
    grid=(GRID,),
    in_specs=[_row, _row, _row, _col, _col, _vec, _mat],
    out_specs=_row,
    out_shape=jax.ShapeDtypeStruct((N_PAD, D), jnp.float32),
)

_final_tc = pl.pallas_call(
    _final_body,
    grid=(GRID,),
    in_specs=[_row, _row, _row, _col, _col, _vec,
              pl.BlockSpec((BN, 1), lambda i: (i, 0))],
    out_specs=pl.BlockSpec((G, D), lambda i: (0, 0)),
    out_shape=jax.ShapeDtypeStruct((G, D), jnp.float32),
    scratch_shapes=[pltpu.VMEM((G, D), jnp.float32),
                    pltpu.VMEM((G, D), jnp.float32)],
    compiler_params=pltpu.CompilerParams(
        dimension_semantics=("arbitrary",)),
)


def _accpad(a):
    return jnp.pad(a, ((0, 0), (0, N_PAD - N_ACC), (0, 0)))


def kernel(x, edge_index, batch, W1, b1, W2, b2, W3, b3):
    src = edge_index[0]
    dst = edge_index[1]
    padn = E_PAD - E
    padr = N + (jnp.arange(padn, dtype=jnp.int32) % (N_ACC - N))
    packed = jnp.concatenate(
        [jnp.left_shift(src, 15) | dst,
         jnp.left_shift(padr, 15) | padr]).reshape(NW, CHUNKS * EC)
    dstp = jnp.concatenate([dst, padr]).reshape(NW, CHUNKS, EC)
    xp = jnp.pad(x, ((0, N_PAD - N), (0, 0)))
    btp = jnp.pad(batch, (0, N_PAD - N),
                  constant_values=G).reshape(N_PAD, 1)
    z1 = jnp.zeros((ROWS_PER,), jnp.float32)
    z2 = jnp.zeros((ROWS_ACC, D), jnp.float32)

    deg_fn, agg_fn = _sc_kernels()
    degs = deg_fn(dstp, z1)
    d0 = degs[0].reshape(N_PAD, 1)
    d1 = degs[1].reshape(N_PAD, 1)

    h1p = _proj_tc(xp, W1, d0, d1)
    a1 = _accpad(agg_fn(h1p, packed, z2))
    h2p = _mid_tc(a1[0], a1[1], h1p, d0, d1, b1.reshape(1, D), W2)
    a2 = _accpad(agg_fn(h2p, packed, z2))
    h3p = _mid_tc(a2[0], a2[1], h2p, d0, d1, b2.reshape(1, D), W3)
    a3 = _accpad(agg_fn(h3p, packed, z2))
    return _final_tc(a3[0], a3[1], h3p, d0, d1, b3.reshape(1, D), btp)
